# Initial kernel scaffold; baseline (speedup 1.0000x reference)
import jax
import jax.numpy as jnp
from jax.experimental import pallas as pl


def kernel(pred, target, pcd_radius):
    raise NotImplementedError("write your pallas kernel here")


# speedup vs baseline: 11.2299x; 11.2299x over previous
import functools

import jax
import jax.numpy as jnp
from jax import lax
from jax.experimental import pallas as pl
from jax.experimental.pallas import tpu as pltpu
from jax.experimental.pallas import tpu_sc as plsc

_ALPHA = 1.0
_RADIUS = 0.07
_H2 = 0.03 * 0.03
_EPS = 1e-12
_B = 8
_N = 2048
_K = 4
_NC, _NS, _L = 2, 16, 16
_NW = _NC * _NS
_RPW = (_B * _N) // _NW
_QPB = _N // _RPW
_GROUPS = _RPW // _L
_MB = _N // _L


def _sqrt16(x):
    i = lax.bitcast_convert_type(x, jnp.int32)
    i = jnp.int32(0x5F3759DF) - lax.shift_right_arithmetic(i, 1)
    y = lax.bitcast_convert_type(i, jnp.float32)
    for _ in range(3):
        y = y * (1.5 - 0.5 * x * y * y)
    return x * y


def _bf16q(x):
    u = lax.bitcast_convert_type(x, jnp.int32)
    half = lax.shift_right_logical(u, 16) & jnp.int32(1)
    u = u + jnp.int32(0x7FFF) + half
    u = u & jnp.int32(-65536)
    return lax.bitcast_convert_type(u, jnp.float32)


def _rep_body(xs, ys, zs, out, xv, yv, zv, accv):
    wid = lax.axis_index("s") * _NC + lax.axis_index("c")
    b = wid // _QPB
    q = wid % _QPB
    pltpu.sync_copy(xs.at[pl.ds(b * _N, _N)], xv)
    pltpu.sync_copy(ys.at[pl.ds(b * _N, _N)], yv)
    pltpu.sync_copy(zs.at[pl.ds(b * _N, _N)], zv)

    inf = jnp.full((_L,), jnp.inf, jnp.float32)

    def group(g, acc):
        r0 = q * _RPW + g * _L
        xn = xv[pl.ds(r0, _L)]
        yn = yv[pl.ds(r0, _L)]
        zn = zv[pl.ds(r0, _L)]
        pn2 = (xn * xn + yn * yn) + zn * zn
        xq = _bf16q(xn)
        yq = _bf16q(yn)
        zq = _bf16q(zn)

        def mblock(mb, st):
            base = mb * _L
            cx = xv[pl.ds(base, _L)]
            cy = yv[pl.ds(base, _L)]
            cz = zv[pl.ds(base, _L)]
            cxq = _bf16q(cx)
            cyq = _bf16q(cy)
            czq = _bf16q(cz)
            pm2 = (cx * cx + cy * cy) + cz * cz
            ks = list(st[0:5])
            vx = list(st[5:10])
            for j in range(_L):
                s = (xq * cxq[j] + yq * cyq[j]) + zq * czq[j]
                kq = (pn2 - (s + s)) + pm2[j]
                dx = xn - cx[j]
                dy = yn - cy[j]
                dz = zn - cz[j]
                vq = (dx * dx + dy * dy) + dz * dz
                for t in range(5):
                    lt = kq < ks[t]
                    nk = jnp.minimum(ks[t], kq)
                    dk = jnp.maximum(ks[t], kq)
                    nv = jnp.where(lt, vq, vx[t])
                    dv = jnp.where(lt, vx[t], vq)
                    ks[t] = nk
                    vx[t] = nv
                    kq = dk
                    vq = dv
            return tuple(ks) + tuple(vx)

        zf = jnp.zeros((_L,), jnp.float32)
        st = lax.fori_loop(0, _MB, mblock,
                           (inf, inf, inf, inf, inf, zf, zf, zf, zf, zf))
        for vt in st[6:10]:
            g2 = jnp.maximum(vt, jnp.float32(_EPS))
            gd = _sqrt16(g2)
            w = jnp.exp(g2 * jnp.float32(-1.0 / _H2))
            acc = acc + (jnp.float32(_RADIUS) - gd) * w
        return acc

    acc = lax.fori_loop(0, _GROUPS, group, jnp.zeros((_L,), jnp.float32))
    accv[...] = acc
    pltpu.sync_copy(accv, out.at[wid])


_rep_partials = functools.partial(
    pl.kernel,
    out_type=jax.ShapeDtypeStruct((_NW, _L), jnp.float32),
    mesh=plsc.VectorSubcoreMesh(core_axis_name="c", subcore_axis_name="s"),
    scratch_types=[
        pltpu.VMEM((_N,), jnp.float32),
        pltpu.VMEM((_N,), jnp.float32),
        pltpu.VMEM((_N,), jnp.float32),
        pltpu.VMEM((_L,), jnp.float32),
    ],
)(_rep_body)


_TR = 512


def _emd_body(pred_blk, tt_blk, out_blk):
    pb = pred_blk[0]
    tt = tt_blk[0]
    pq = pb.astype(jnp.bfloat16)
    tq = tt.astype(jnp.bfloat16)
    innerq = lax.dot_general(pq, tq, (((1,), (0,)), ((), ())),
                             preferred_element_type=jnp.float32)
    pn2 = jnp.sum(pb * pb, axis=1, keepdims=True)
    tm2 = jnp.sum(tt * tt, axis=0, keepdims=True)
    d2q = (pn2 - 2.0 * innerq) + tm2
    dx = pb[:, 0:1] - tt[0:1, :]
    dy = pb[:, 1:2] - tt[1:2, :]
    dz = pb[:, 2:3] - tt[2:3, :]
    d2x = (dx * dx + dy * dy) + dz * dz
    rowminq = jnp.min(d2q, axis=1, keepdims=True)
    rowval = jnp.min(jnp.where(d2q == rowminq, d2x, jnp.inf), axis=1)
    out_blk[pl.program_id(0), pl.program_id(1)] = jnp.sum(rowval)


def _emd_partials(pred, target_t):
    return pl.pallas_call(
        _emd_body,
        grid=(_B, _N // _TR),
        in_specs=[
            pl.BlockSpec((1, _TR, 3), lambda b, i: (b, i, 0)),
            pl.BlockSpec((1, 3, _N), lambda b, i: (b, 0, 0)),
        ],
        out_specs=pl.BlockSpec((_B, _N // _TR), lambda b, i: (0, 0),
                               memory_space=pltpu.SMEM),
        out_shape=jax.ShapeDtypeStruct((_B, _N // _TR), jnp.float32),
    )(pred, target_t)


def kernel(pred, target, pcd_radius):
    target_t = jnp.transpose(target, (0, 2, 1))
    xs = jnp.reshape(pred[:, :, 0], (-1,))
    ys = jnp.reshape(pred[:, :, 1], (-1,))
    zs = jnp.reshape(pred[:, :, 2], (-1,))
    rep = _rep_partials(xs, ys, zs)
    emd_p = _emd_partials(pred, target_t)
    dist2 = jnp.sum(emd_p, axis=1, keepdims=True) / jnp.float32(_N * 3)
    emd_loss = 100.0 * jnp.mean(dist2 / pcd_radius)
    uniform_loss = _ALPHA * (jnp.sum(rep) / jnp.float32(_B * _N * _K))
    return (emd_loss, uniform_loss)


# speedup vs baseline: 11.2610x; 1.0028x over previous
import functools

import jax
import jax.numpy as jnp
from jax import lax
from jax.experimental import pallas as pl
from jax.experimental.pallas import tpu as pltpu
from jax.experimental.pallas import tpu_sc as plsc

_ALPHA = 1.0
_RADIUS = 0.07
_H2 = 0.03 * 0.03
_EPS = 1e-12
_B = 8
_N = 2048
_K = 4
_NC, _NS, _L = 2, 16, 16
_NW = _NC * _NS
_RPW = (_B * _N) // _NW
_QPB = _N // _RPW
_GROUPS = _RPW // _L
_MB = _N // _L


def _sqrt16(x):
    i = lax.bitcast_convert_type(x, jnp.int32)
    i = jnp.int32(0x5F3759DF) - lax.shift_right_arithmetic(i, 1)
    y = lax.bitcast_convert_type(i, jnp.float32)
    for _ in range(3):
        y = y * (1.5 - 0.5 * x * y * y)
    return x * y


def _bf16q(x):
    u = lax.bitcast_convert_type(x, jnp.int32)
    half = lax.shift_right_logical(u, 16) & jnp.int32(1)
    u = u + jnp.int32(0x7FFF) + half
    u = u & jnp.int32(-65536)
    return lax.bitcast_convert_type(u, jnp.float32)


def _rep_body(xs, ys, zs, out, xv, yv, zv, accv):
    wid = lax.axis_index("s") * _NC + lax.axis_index("c")
    b = wid // _QPB
    q = wid % _QPB
    pltpu.sync_copy(xs.at[pl.ds(b * _N, _N)], xv)
    pltpu.sync_copy(ys.at[pl.ds(b * _N, _N)], yv)
    pltpu.sync_copy(zs.at[pl.ds(b * _N, _N)], zv)

    inf = jnp.full((_L,), jnp.inf, jnp.float32)

    def group(g, acc):
        r0 = q * _RPW + g * _L
        xn = xv[pl.ds(r0, _L)]
        yn = yv[pl.ds(r0, _L)]
        zn = zv[pl.ds(r0, _L)]
        pn2 = (xn * xn + yn * yn) + zn * zn
        xq = _bf16q(xn)
        yq = _bf16q(yn)
        zq = _bf16q(zn)

        def insert(ks, vx, kq, vq, last_full):
            for t in range(5):
                lt = kq < ks[t]
                nk = jnp.minimum(ks[t], kq)
                nv = jnp.where(lt, vq, vx[t])
                if t < 4 or last_full:
                    dk = jnp.maximum(ks[t], kq)
                    dv = jnp.where(lt, vx[t], vq)
                    kq = dk
                    vq = dv
                ks[t] = nk
                vx[t] = nv
            return ks, vx

        def mblock(mb, st):
            base = mb * _L
            cx = xv[pl.ds(base, _L)]
            cy = yv[pl.ds(base, _L)]
            cz = zv[pl.ds(base, _L)]
            cxq = _bf16q(cx)
            cyq = _bf16q(cy)
            czq = _bf16q(cz)
            pm2 = (cx * cx + cy * cy) + cz * cz
            ka = list(st[0:5])
            va = list(st[5:10])
            kb = list(st[10:15])
            vb = list(st[15:20])
            for j in range(_L):
                s = (xq * cxq[j] + yq * cyq[j]) + zq * czq[j]
                kq = (pn2 - (s + s)) + pm2[j]
                dx = xn - cx[j]
                dy = yn - cy[j]
                dz = zn - cz[j]
                vq = (dx * dx + dy * dy) + dz * dz
                if j % 2 == 0:
                    ka, va = insert(ka, va, kq, vq, False)
                else:
                    kb, vb = insert(kb, vb, kq, vq, False)
            return tuple(ka) + tuple(va) + tuple(kb) + tuple(vb)

        zf = jnp.zeros((_L,), jnp.float32)
        st = lax.fori_loop(0, _MB, mblock,
                           (inf, inf, inf, inf, inf, zf, zf, zf, zf, zf) * 2)
        ka = list(st[0:5])
        va = list(st[5:10])
        for t in range(5):
            ka, va = insert(ka, va, st[10 + t], st[15 + t], False)
        for vt in va[1:5]:
            g2 = jnp.maximum(vt, jnp.float32(_EPS))
            gd = _sqrt16(g2)
            w = jnp.exp(g2 * jnp.float32(-1.0 / _H2))
            acc = acc + (jnp.float32(_RADIUS) - gd) * w
        return acc

    acc = lax.fori_loop(0, _GROUPS, group, jnp.zeros((_L,), jnp.float32))
    accv[...] = acc
    pltpu.sync_copy(accv, out.at[wid])


_rep_partials = functools.partial(
    pl.kernel,
    out_type=jax.ShapeDtypeStruct((_NW, _L), jnp.float32),
    mesh=plsc.VectorSubcoreMesh(core_axis_name="c", subcore_axis_name="s"),
    scratch_types=[
        pltpu.VMEM((_N,), jnp.float32),
        pltpu.VMEM((_N,), jnp.float32),
        pltpu.VMEM((_N,), jnp.float32),
        pltpu.VMEM((_L,), jnp.float32),
    ],
)(_rep_body)


_TR = 512


def _emd_body(pred_blk, tt_blk, out_blk):
    pb = pred_blk[0]
    tt = tt_blk[0]
    pq = pb.astype(jnp.bfloat16)
    tq = tt.astype(jnp.bfloat16)
    innerq = lax.dot_general(pq, tq, (((1,), (0,)), ((), ())),
                             preferred_element_type=jnp.float32)
    pn2 = jnp.sum(pb * pb, axis=1, keepdims=True)
    tm2 = jnp.sum(tt * tt, axis=0, keepdims=True)
    d2q = (pn2 - 2.0 * innerq) + tm2
    dx = pb[:, 0:1] - tt[0:1, :]
    dy = pb[:, 1:2] - tt[1:2, :]
    dz = pb[:, 2:3] - tt[2:3, :]
    d2x = (dx * dx + dy * dy) + dz * dz
    rowminq = jnp.min(d2q, axis=1, keepdims=True)
    rowval = jnp.min(jnp.where(d2q == rowminq, d2x, jnp.inf), axis=1)
    out_blk[pl.program_id(0), pl.program_id(1)] = jnp.sum(rowval)


def _emd_partials(pred, target_t):
    return pl.pallas_call(
        _emd_body,
        grid=(_B, _N // _TR),
        in_specs=[
            pl.BlockSpec((1, _TR, 3), lambda b, i: (b, i, 0)),
            pl.BlockSpec((1, 3, _N), lambda b, i: (b, 0, 0)),
        ],
        out_specs=pl.BlockSpec((_B, _N // _TR), lambda b, i: (0, 0),
                               memory_space=pltpu.SMEM),
        out_shape=jax.ShapeDtypeStruct((_B, _N // _TR), jnp.float32),
    )(pred, target_t)


def kernel(pred, target, pcd_radius):
    target_t = jnp.transpose(target, (0, 2, 1))
    xs = jnp.reshape(pred[:, :, 0], (-1,))
    ys = jnp.reshape(pred[:, :, 1], (-1,))
    zs = jnp.reshape(pred[:, :, 2], (-1,))
    rep = _rep_partials(xs, ys, zs)
    emd_p = _emd_partials(pred, target_t)
    dist2 = jnp.sum(emd_p, axis=1, keepdims=True) / jnp.float32(_N * 3)
    emd_loss = 100.0 * jnp.mean(dist2 / pcd_radius)
    uniform_loss = _ALPHA * (jnp.sum(rep) / jnp.float32(_B * _N * _K))
    return (emd_loss, uniform_loss)
